# baseline (speedup 1.0000x reference)
import functools

import jax
import jax.numpy as jnp
from jax import lax
from jax.experimental import pallas as pl
from jax.experimental.pallas import tpu as pltpu
from jax.experimental.pallas import tpu_sc as plsc

N = 10000
E = 320000
D = 128

NC = 2
NS = 16
NW = NC * NS

CH = 80
CPW = 125

ACH = 128
ACPW = 80
HCH = ACPW // 2
EPAD = NW * ACPW * ACH
NTRASH = 480
NACC = N + NTRASH
RPA = 656
RPL = NACC - 15 * RPA
CPL = N - 15 * RPA

NPAD = 10240
RPT = NPAD // NS

_mesh = plsc.VectorSubcoreMesh(
    core_axis_name="c", subcore_axis_name="s", num_cores=NC, num_subcores=NS
)


@functools.partial(
    pl.kernel,
    out_type=jax.ShapeDtypeStruct((NC, 2, NPAD), jnp.float32),
    mesh=_mesh,
    scratch_types=[
        pltpu.VMEM((CPW, CH), jnp.int32),
        pltpu.VMEM((CPW, CH), jnp.int32),
        pltpu.VMEM((RPT,), jnp.float32),
        pltpu.VMEM((CH,), jnp.float32),
        pltpu.VMEM_SHARED((NPAD,), jnp.float32),
        pltpu.VMEM_SHARED((NPAD,), jnp.float32),
        pltpu.SemaphoreType.DMA,
        pltpu.SemaphoreType.DMA,
    ],
)
def _deg_sc(src_hbm, dst_hbm, out_hbm, src_v, dst_v, zbuf, ones_v,
            dego_acc, degi_acc, sd, si):
    c = lax.axis_index("c")
    s = lax.axis_index("s")
    w = c * NS + s
    z16 = jnp.zeros((16,), jnp.float32)
    o16 = jnp.ones((16,), jnp.float32)

    hs = pltpu.async_copy(src_hbm.at[w], src_v, si)
    hd = pltpu.async_copy(dst_hbm.at[w], dst_v, si)

    def zrow(i, carry):
        zbuf[pl.ds(i * 16, 16)] = z16
        return carry
    lax.fori_loop(0, RPT // 16, zrow, 0)
    for q in range(CH // 16):
        ones_v[pl.ds(q * 16, 16)] = o16

    pltpu.sync_copy(zbuf, dego_acc.at[pl.ds(s * RPT, RPT)])
    pltpu.sync_copy(zbuf, degi_acc.at[pl.ds(s * RPT, RPT)])
    hs.wait()
    hd.wait()
    plsc.subcore_barrier()

    def group(g, carry):
        for b in range(25):
            j = 25 * g + b
            pltpu.async_copy(ones_v, dego_acc.at[src_v.at[j]], sd, add=True)
            pltpu.async_copy(ones_v, degi_acc.at[dst_v.at[j]], sd, add=True)
        for b in range(25):
            j = 25 * g + b
            pltpu.make_async_copy(ones_v, dego_acc.at[src_v.at[j]],
                                  sd).wait()
            pltpu.make_async_copy(ones_v, degi_acc.at[dst_v.at[j]],
                                  sd).wait()
        return carry
    lax.fori_loop(0, CPW // 25, group, 0)
    plsc.subcore_barrier()

    pltpu.sync_copy(dego_acc.at[pl.ds(s * RPT, RPT)],
                    out_hbm.at[c, 0, pl.ds(s * RPT, RPT)])
    pltpu.sync_copy(degi_acc.at[pl.ds(s * RPT, RPT)],
                    out_hbm.at[c, 1, pl.ds(s * RPT, RPT)])


@functools.partial(
    pl.kernel,
    out_type=jax.ShapeDtypeStruct((NC, N, D), jnp.float32),
    mesh=_mesh,
    scratch_types=[
        pltpu.VMEM((HCH, ACH), jnp.int32),
        pltpu.VMEM((HCH, ACH), jnp.int32),
        pltpu.VMEM((2, ACH, D), jnp.float32),
        pltpu.VMEM_SHARED((NACC, D), jnp.float32),
        pltpu.SemaphoreType.DMA,
        pltpu.SemaphoreType.DMA,
        pltpu.SemaphoreType.DMA,
        pltpu.SemaphoreType.DMA,
        pltpu.SemaphoreType.DMA,
    ],
)
def _agg_sc(table_hbm, src_hbm, dst_hbm, zeros_hbm, out_hbm, src_v, dst_v,
            rows_v, acc, sg0, sg1, ss0, ss1, si):
    c = lax.axis_index("c")
    s = lax.axis_index("s")
    w = c * NS + s
    sg = (sg0, sg1)
    ss = (ss0, ss1)

    hs = pltpu.async_copy(src_hbm.at[w, pl.ds(0, HCH), :], src_v, si)
    hd = pltpu.async_copy(dst_hbm.at[w, pl.ds(0, HCH), :], dst_v, si)
    base = s * RPA

    @pl.when(s < NS - 1)
    def _():
        pltpu.async_copy(zeros_hbm.at[pl.ds(base, RPA), :],
                         acc.at[pl.ds(base, RPA), :], si)

    @pl.when(s == NS - 1)
    def _():
        pltpu.async_copy(zeros_hbm.at[pl.ds(base, RPL), :],
                         acc.at[pl.ds(base, RPL), :], si)
    hs.wait()
    hd.wait()

    @pl.when(s < NS - 1)
    def _():
        pltpu.make_async_copy(zeros_hbm.at[pl.ds(base, RPA), :],
                              acc.at[pl.ds(base, RPA), :], si).wait()

    @pl.when(s == NS - 1)
    def _():
        pltpu.make_async_copy(zeros_hbm.at[pl.ds(base, RPL), :],
                              acc.at[pl.ds(base, RPL), :], si).wait()
    plsc.subcore_barrier()

    def gather(j, b):
        pltpu.async_copy(table_hbm.at[src_v.at[j]], rows_v.at[b], sg[b])

    def gather_wait(j, b):
        pltpu.make_async_copy(table_hbm.at[src_v.at[j]], rows_v.at[b],
                              sg[b]).wait()

    def scatter(j, b):
        pltpu.async_copy(rows_v.at[b], acc.at[dst_v.at[j]], ss[b],
                         add=True)

    def scatter_wait(j, b):
        pltpu.make_async_copy(rows_v.at[b], acc.at[dst_v.at[j]],
                              ss[b]).wait()

    for p in range(2):
        if p == 1:
            pltpu.sync_copy(src_hbm.at[w, pl.ds(HCH, HCH), :], src_v)
            pltpu.sync_copy(dst_hbm.at[w, pl.ds(HCH, HCH), :], dst_v)
        gather(0, 0)

        def group(g, carry):
            for b in range(2):
                j = 2 * g + b
                gather_wait(j, b)
                scatter(j, b)

                @pl.when(j > 0)
                def _():
                    scatter_wait(j - 1, 1 - b)

                @pl.when(j < HCH - 1)
                def _():
                    gather(j + 1, 1 - b)
            return carry
        lax.fori_loop(0, HCH // 2, group, 0)
        scatter_wait(HCH - 1, 1)
    plsc.subcore_barrier()

    @pl.when(s < NS - 1)
    def _():
        pltpu.sync_copy(acc.at[pl.ds(s * RPA, RPA), :],
                        out_hbm.at[c, pl.ds(s * RPA, RPA), :])

    @pl.when(s == NS - 1)
    def _():
        pltpu.sync_copy(acc.at[pl.ds(s * RPA, CPL), :],
                        out_hbm.at[c, pl.ds(s * RPA, CPL), :])


_B = 1024


def _rs(col):
    return lax.rsqrt(jnp.maximum(col, 1.0))


def _t1_body(x_ref, w_ref, b_ref, dg_ref, o_ref):
    dg = dg_ref[...]
    rs_out = _rs(dg[:, 0:1] + dg[:, 2:3])
    o_ref[...] = (jnp.dot(x_ref[...], w_ref[...],
                          preferred_element_type=jnp.float32)
                  + b_ref[...]) * rs_out


def _t2_body(a_ref, w_ref, b_ref, dg_ref, o_ref):
    dg = dg_ref[...]
    rs_in = _rs(dg[:, 1:2] + dg[:, 3:4])
    rs_out = _rs(dg[:, 0:1] + dg[:, 2:3])
    h = jnp.maximum((a_ref[0] + a_ref[1]) * rs_in, 0.0)
    o_ref[...] = (jnp.dot(h, w_ref[...],
                          preferred_element_type=jnp.float32)
                  + b_ref[...]) * rs_out


def _t3_body(a_ref, dg_ref, f1_ref, g1_ref, f2_ref, g2_ref, f3_ref, g3_ref,
             fs_ref, gs_ref, o_ref):
    dg = dg_ref[...]
    rs_in = _rs(dg[:, 1:2] + dg[:, 3:4])
    h = (a_ref[0] + a_ref[1]) * rs_in
    ff = jnp.maximum(jnp.dot(h, f1_ref[...],
                             preferred_element_type=jnp.float32)
                     + g1_ref[...], 0.0)
    ff = jnp.maximum(jnp.dot(ff, f2_ref[...],
                             preferred_element_type=jnp.float32)
                     + g2_ref[...], 0.0)
    ff = jnp.maximum(jnp.dot(ff, f3_ref[...],
                             preferred_element_type=jnp.float32)
                     + g3_ref[...], 0.0)
    o_ref[...] = ff + jnp.dot(h, fs_ref[...],
                              preferred_element_type=jnp.float32) + gs_ref[...]


def _row_spec():
    return pl.BlockSpec((_B, D), lambda i: (i, 0))


def _full_spec(shape):
    return pl.BlockSpec(shape, lambda i: tuple(0 for _ in shape))


def _dg_spec():
    return pl.BlockSpec((_B, 4), lambda i: (i, 0))


def _agg_spec():
    return pl.BlockSpec((NC, _B, D), lambda i: (0, i, 0))


_GRID = ((N + _B - 1) // _B,)


def _t1(x, W, b, degT):
    return pl.pallas_call(
        _t1_body,
        grid=_GRID,
        in_specs=[_row_spec(), _full_spec((D, D)), _full_spec((1, D)),
                  _dg_spec()],
        out_specs=_row_spec(),
        out_shape=jax.ShapeDtypeStruct((N, D), jnp.float32),
    )(x, W, b, degT)


def _t2(a, W, b, degT):
    return pl.pallas_call(
        _t2_body,
        grid=_GRID,
        in_specs=[_agg_spec(), _full_spec((D, D)), _full_spec((1, D)),
                  _dg_spec()],
        out_specs=_row_spec(),
        out_shape=jax.ShapeDtypeStruct((N, D), jnp.float32),
    )(a, W, b, degT)


def _t3(a, degT, F1, g1, F2, g2, F3, g3, Fs, gs):
    fw = _full_spec((D, D))
    fb = _full_spec((1, D))
    return pl.pallas_call(
        _t3_body,
        grid=_GRID,
        in_specs=[_agg_spec(), _dg_spec(), fw, fb, fw, fb, fw, fb, fw, fb],
        out_specs=_row_spec(),
        out_shape=jax.ShapeDtypeStruct((N, D), jnp.float32),
    )(a, degT, F1, g1, F2, g2, F3, g3, Fs, gs)


def kernel(x, edge_index, W1, b1, W2, b2, F1, fb1, F2, fb2, F3, fb3, Fs, fbs):
    src_d = edge_index[0].reshape(NW, CPW, CH)
    dst_d = edge_index[1].reshape(NW, CPW, CH)
    npd = EPAD - E
    pad_src = (jnp.arange(npd, dtype=jnp.int32) % N)
    pad_dst = N + (jnp.arange(npd, dtype=jnp.int32) % NTRASH)
    src_a = jnp.concatenate([edge_index[0], pad_src]).reshape(NW, ACPW, ACH)
    dst_a = jnp.concatenate([edge_index[1], pad_dst]).reshape(NW, ACPW, ACH)

    zrows = jnp.zeros((NACC, D), jnp.float32)

    deg = _deg_sc(src_d, dst_d)
    degT = jnp.transpose(deg, (2, 0, 1)).reshape(NPAD, 4)

    s1 = _t1(x, W1, b1.reshape(1, D), degT)
    a1 = _agg_sc(s1, src_a, dst_a, zrows)
    s2 = _t2(a1, W2, b2.reshape(1, D), degT)
    a2 = _agg_sc(s2, src_a, dst_a, zrows)
    return _t3(a2, degT, F1, fb1.reshape(1, D), F2, fb2.reshape(1, D),
               F3, fb3.reshape(1, D), Fs, fbs.reshape(1, D))


# speedup vs baseline: 1.0018x; 1.0018x over previous
import functools

import jax
import jax.numpy as jnp
from jax import lax
from jax.experimental import pallas as pl
from jax.experimental.pallas import tpu as pltpu
from jax.experimental.pallas import tpu_sc as plsc

N = 10000
E = 320000
D = 128

NC = 2
NS = 16
NW = NC * NS

CH = 80
CPW = 125

ACH = 128
ACPW = 80
HCH = ACPW // 2
EPAD = NW * ACPW * ACH
NTRASH = 480
NACC = N + NTRASH
RPA = 656
RPL = NACC - 15 * RPA
CPL = N - 15 * RPA

NPAD = 10240
RPT = NPAD // NS

_mesh = plsc.VectorSubcoreMesh(
    core_axis_name="c", subcore_axis_name="s", num_cores=NC, num_subcores=NS
)


@functools.partial(
    pl.kernel,
    out_type=jax.ShapeDtypeStruct((NC, 2, NPAD), jnp.float32),
    mesh=_mesh,
    scratch_types=[
        pltpu.VMEM((CPW, CH), jnp.int32),
        pltpu.VMEM((CPW, CH), jnp.int32),
        pltpu.VMEM((RPT,), jnp.float32),
        pltpu.VMEM((CH,), jnp.float32),
        pltpu.VMEM_SHARED((NPAD,), jnp.float32),
        pltpu.VMEM_SHARED((NPAD,), jnp.float32),
        pltpu.SemaphoreType.DMA,
        pltpu.SemaphoreType.DMA,
    ],
)
def _deg_sc(src_hbm, dst_hbm, out_hbm, src_v, dst_v, zbuf, ones_v,
            dego_acc, degi_acc, sd, si):
    c = lax.axis_index("c")
    s = lax.axis_index("s")
    w = c * NS + s
    z16 = jnp.zeros((16,), jnp.float32)
    o16 = jnp.ones((16,), jnp.float32)

    hs = pltpu.async_copy(src_hbm.at[w], src_v, si)
    hd = pltpu.async_copy(dst_hbm.at[w], dst_v, si)

    def zrow(i, carry):
        zbuf[pl.ds(i * 16, 16)] = z16
        return carry
    lax.fori_loop(0, RPT // 16, zrow, 0)
    for q in range(CH // 16):
        ones_v[pl.ds(q * 16, 16)] = o16

    pltpu.sync_copy(zbuf, dego_acc.at[pl.ds(s * RPT, RPT)])
    pltpu.sync_copy(zbuf, degi_acc.at[pl.ds(s * RPT, RPT)])
    hs.wait()
    hd.wait()
    plsc.subcore_barrier()

    def group(g, carry):
        for b in range(25):
            j = 25 * g + b
            pltpu.async_copy(ones_v, dego_acc.at[src_v.at[j]], sd, add=True)
            pltpu.async_copy(ones_v, degi_acc.at[dst_v.at[j]], sd, add=True)
        for b in range(25):
            j = 25 * g + b
            pltpu.make_async_copy(ones_v, dego_acc.at[src_v.at[j]],
                                  sd).wait()
            pltpu.make_async_copy(ones_v, degi_acc.at[dst_v.at[j]],
                                  sd).wait()
        return carry
    lax.fori_loop(0, CPW // 25, group, 0)
    plsc.subcore_barrier()

    pltpu.sync_copy(dego_acc.at[pl.ds(s * RPT, RPT)],
                    out_hbm.at[c, 0, pl.ds(s * RPT, RPT)])
    pltpu.sync_copy(degi_acc.at[pl.ds(s * RPT, RPT)],
                    out_hbm.at[c, 1, pl.ds(s * RPT, RPT)])


@functools.partial(
    pl.kernel,
    out_type=jax.ShapeDtypeStruct((NC, N, D), jnp.float32),
    mesh=_mesh,
    scratch_types=[
        pltpu.VMEM((HCH, ACH), jnp.int32),
        pltpu.VMEM((HCH, ACH), jnp.int32),
        pltpu.VMEM((2, ACH, D), jnp.float32),
        pltpu.VMEM_SHARED((NACC, D), jnp.float32),
        pltpu.SemaphoreType.DMA,
        pltpu.SemaphoreType.DMA,
        pltpu.SemaphoreType.DMA,
        pltpu.SemaphoreType.DMA,
        pltpu.SemaphoreType.DMA,
    ],
)
def _agg_sc(table_hbm, src_hbm, dst_hbm, zeros_hbm, out_hbm, src_v, dst_v,
            rows_v, acc, sg0, sg1, ss0, ss1, si):
    c = lax.axis_index("c")
    s = lax.axis_index("s")
    w = c * NS + s
    sg = (sg0, sg1)
    ss = (ss0, ss1)

    hs = pltpu.async_copy(src_hbm.at[w, pl.ds(0, HCH), :], src_v, si)
    hd = pltpu.async_copy(dst_hbm.at[w, pl.ds(0, HCH), :], dst_v, si)
    base = s * RPA

    @pl.when(s < NS - 1)
    def _():
        pltpu.async_copy(zeros_hbm.at[pl.ds(base, RPA), :],
                         acc.at[pl.ds(base, RPA), :], si)

    @pl.when(s == NS - 1)
    def _():
        pltpu.async_copy(zeros_hbm.at[pl.ds(base, RPL), :],
                         acc.at[pl.ds(base, RPL), :], si)
    hs.wait()
    hd.wait()

    @pl.when(s < NS - 1)
    def _():
        pltpu.make_async_copy(zeros_hbm.at[pl.ds(base, RPA), :],
                              acc.at[pl.ds(base, RPA), :], si).wait()

    @pl.when(s == NS - 1)
    def _():
        pltpu.make_async_copy(zeros_hbm.at[pl.ds(base, RPL), :],
                              acc.at[pl.ds(base, RPL), :], si).wait()
    plsc.subcore_barrier()

    def gather(j, b):
        pltpu.async_copy(table_hbm.at[src_v.at[j]], rows_v.at[b], sg[b])

    def gather_wait(j, b):
        pltpu.make_async_copy(table_hbm.at[src_v.at[j]], rows_v.at[b],
                              sg[b]).wait()

    def scatter(j, b):
        pltpu.async_copy(rows_v.at[b], acc.at[dst_v.at[j]], ss[b],
                         add=True)

    def scatter_wait(j, b):
        pltpu.make_async_copy(rows_v.at[b], acc.at[dst_v.at[j]],
                              ss[b]).wait()

    for p in range(2):
        if p == 1:
            pltpu.sync_copy(src_hbm.at[w, pl.ds(HCH, HCH), :], src_v)
            pltpu.sync_copy(dst_hbm.at[w, pl.ds(HCH, HCH), :], dst_v)
        gather(0, 0)
        gather_wait(0, 0)
        scatter(0, 0)
        gather(1, 1)

        def group(g, carry):
            for b in range(2):
                j = 2 * g + 1 + b
                gather_wait(j, 1 - b)
                scatter(j, 1 - b)
                scatter_wait(j - 1, b)
                gather(j + 1, b)
            return carry
        lax.fori_loop(0, (HCH - 2) // 2, group, 0)
        gather_wait(HCH - 1, 1)
        scatter(HCH - 1, 1)
        scatter_wait(HCH - 2, 0)
        scatter_wait(HCH - 1, 1)
    plsc.subcore_barrier()

    @pl.when(s < NS - 1)
    def _():
        pltpu.sync_copy(acc.at[pl.ds(s * RPA, RPA), :],
                        out_hbm.at[c, pl.ds(s * RPA, RPA), :])

    @pl.when(s == NS - 1)
    def _():
        pltpu.sync_copy(acc.at[pl.ds(s * RPA, CPL), :],
                        out_hbm.at[c, pl.ds(s * RPA, CPL), :])


_B = 1024


def _rs(col):
    return lax.rsqrt(jnp.maximum(col, 1.0))


def _t1_body(x_ref, w_ref, b_ref, dg_ref, o_ref):
    dg = dg_ref[...]
    rs_out = _rs(dg[:, 0:1] + dg[:, 2:3])
    o_ref[...] = (jnp.dot(x_ref[...], w_ref[...],
                          preferred_element_type=jnp.float32)
                  + b_ref[...]) * rs_out


def _t2_body(a_ref, w_ref, b_ref, dg_ref, o_ref):
    dg = dg_ref[...]
    rs_in = _rs(dg[:, 1:2] + dg[:, 3:4])
    rs_out = _rs(dg[:, 0:1] + dg[:, 2:3])
    h = jnp.maximum((a_ref[0] + a_ref[1]) * rs_in, 0.0)
    o_ref[...] = (jnp.dot(h, w_ref[...],
                          preferred_element_type=jnp.float32)
                  + b_ref[...]) * rs_out


def _t3_body(a_ref, dg_ref, f1_ref, g1_ref, f2_ref, g2_ref, f3_ref, g3_ref,
             fs_ref, gs_ref, o_ref):
    dg = dg_ref[...]
    rs_in = _rs(dg[:, 1:2] + dg[:, 3:4])
    h = (a_ref[0] + a_ref[1]) * rs_in
    ff = jnp.maximum(jnp.dot(h, f1_ref[...],
                             preferred_element_type=jnp.float32)
                     + g1_ref[...], 0.0)
    ff = jnp.maximum(jnp.dot(ff, f2_ref[...],
                             preferred_element_type=jnp.float32)
                     + g2_ref[...], 0.0)
    ff = jnp.maximum(jnp.dot(ff, f3_ref[...],
                             preferred_element_type=jnp.float32)
                     + g3_ref[...], 0.0)
    o_ref[...] = ff + jnp.dot(h, fs_ref[...],
                              preferred_element_type=jnp.float32) + gs_ref[...]


def _row_spec():
    return pl.BlockSpec((_B, D), lambda i: (i, 0))


def _full_spec(shape):
    return pl.BlockSpec(shape, lambda i: tuple(0 for _ in shape))


def _dg_spec():
    return pl.BlockSpec((_B, 4), lambda i: (i, 0))


def _agg_spec():
    return pl.BlockSpec((NC, _B, D), lambda i: (0, i, 0))


_GRID = ((N + _B - 1) // _B,)


def _t1(x, W, b, degT):
    return pl.pallas_call(
        _t1_body,
        grid=_GRID,
        in_specs=[_row_spec(), _full_spec((D, D)), _full_spec((1, D)),
                  _dg_spec()],
        out_specs=_row_spec(),
        out_shape=jax.ShapeDtypeStruct((N, D), jnp.float32),
    )(x, W, b, degT)


def _t2(a, W, b, degT):
    return pl.pallas_call(
        _t2_body,
        grid=_GRID,
        in_specs=[_agg_spec(), _full_spec((D, D)), _full_spec((1, D)),
                  _dg_spec()],
        out_specs=_row_spec(),
        out_shape=jax.ShapeDtypeStruct((N, D), jnp.float32),
    )(a, W, b, degT)


def _t3(a, degT, F1, g1, F2, g2, F3, g3, Fs, gs):
    fw = _full_spec((D, D))
    fb = _full_spec((1, D))
    return pl.pallas_call(
        _t3_body,
        grid=_GRID,
        in_specs=[_agg_spec(), _dg_spec(), fw, fb, fw, fb, fw, fb, fw, fb],
        out_specs=_row_spec(),
        out_shape=jax.ShapeDtypeStruct((N, D), jnp.float32),
    )(a, degT, F1, g1, F2, g2, F3, g3, Fs, gs)


def kernel(x, edge_index, W1, b1, W2, b2, F1, fb1, F2, fb2, F3, fb3, Fs, fbs):
    src_d = edge_index[0].reshape(NW, CPW, CH)
    dst_d = edge_index[1].reshape(NW, CPW, CH)
    npd = EPAD - E
    pad_src = (jnp.arange(npd, dtype=jnp.int32) % N)
    pad_dst = N + (jnp.arange(npd, dtype=jnp.int32) % NTRASH)
    src_a = jnp.concatenate([edge_index[0], pad_src]).reshape(NW, ACPW, ACH)
    dst_a = jnp.concatenate([edge_index[1], pad_dst]).reshape(NW, ACPW, ACH)

    zrows = jnp.zeros((NACC, D), jnp.float32)

    deg = _deg_sc(src_d, dst_d)
    degT = jnp.transpose(deg, (2, 0, 1)).reshape(NPAD, 4)

    s1 = _t1(x, W1, b1.reshape(1, D), degT)
    a1 = _agg_sc(s1, src_a, dst_a, zrows)
    s2 = _t2(a1, W2, b2.reshape(1, D), degT)
    a2 = _agg_sc(s2, src_a, dst_a, zrows)
    return _t3(a2, degT, F1, fb1.reshape(1, D), F2, fb2.reshape(1, D),
               F3, fb3.reshape(1, D), Fs, fbs.reshape(1, D))


# speedup vs baseline: 1.0189x; 1.0170x over previous
import functools

import jax
import jax.numpy as jnp
from jax import lax
from jax.experimental import pallas as pl
from jax.experimental.pallas import tpu as pltpu
from jax.experimental.pallas import tpu_sc as plsc

N = 10000
E = 320000
D = 128

NC = 2
NS = 16
NW = NC * NS

CH = 80
CPW = 125

ACH = 128
ACPW = 80
HCH = ACPW // 2
EPAD = NW * ACPW * ACH
NTRASH = 480
NACC = N + NTRASH
RPA = 656
RPL = NACC - 15 * RPA
CPL = N - 15 * RPA

NPAD = 10240
RPT = NPAD // NS

_mesh = plsc.VectorSubcoreMesh(
    core_axis_name="c", subcore_axis_name="s", num_cores=NC, num_subcores=NS
)


@functools.partial(
    pl.kernel,
    out_type=jax.ShapeDtypeStruct((NC, 2, NPAD), jnp.float32),
    mesh=_mesh,
    scratch_types=[
        pltpu.VMEM((CPW, CH), jnp.int32),
        pltpu.VMEM((CPW, CH), jnp.int32),
        pltpu.VMEM((RPT,), jnp.float32),
        pltpu.VMEM((CH,), jnp.float32),
        pltpu.VMEM_SHARED((NPAD,), jnp.float32),
        pltpu.VMEM_SHARED((NPAD,), jnp.float32),
        pltpu.SemaphoreType.DMA,
        pltpu.SemaphoreType.DMA,
    ],
)
def _deg_sc(src_hbm, dst_hbm, out_hbm, src_v, dst_v, zbuf, ones_v,
            dego_acc, degi_acc, sd, si):
    c = lax.axis_index("c")
    s = lax.axis_index("s")
    w = c * NS + s
    z16 = jnp.zeros((16,), jnp.float32)
    o16 = jnp.ones((16,), jnp.float32)

    hs = pltpu.async_copy(src_hbm.at[w], src_v, si)
    hd = pltpu.async_copy(dst_hbm.at[w], dst_v, si)

    def zrow(i, carry):
        zbuf[pl.ds(i * 16, 16)] = z16
        return carry
    lax.fori_loop(0, RPT // 16, zrow, 0)
    for q in range(CH // 16):
        ones_v[pl.ds(q * 16, 16)] = o16

    pltpu.sync_copy(zbuf, dego_acc.at[pl.ds(s * RPT, RPT)])
    pltpu.sync_copy(zbuf, degi_acc.at[pl.ds(s * RPT, RPT)])
    hs.wait()
    hd.wait()
    plsc.subcore_barrier()

    def group(g, carry):
        for b in range(25):
            j = 25 * g + b
            pltpu.async_copy(ones_v, dego_acc.at[src_v.at[j]], sd, add=True)
            pltpu.async_copy(ones_v, degi_acc.at[dst_v.at[j]], sd, add=True)
        for b in range(25):
            j = 25 * g + b
            pltpu.make_async_copy(ones_v, dego_acc.at[src_v.at[j]],
                                  sd).wait()
            pltpu.make_async_copy(ones_v, degi_acc.at[dst_v.at[j]],
                                  sd).wait()
        return carry
    lax.fori_loop(0, CPW // 25, group, 0)
    plsc.subcore_barrier()

    pltpu.sync_copy(dego_acc.at[pl.ds(s * RPT, RPT)],
                    out_hbm.at[c, 0, pl.ds(s * RPT, RPT)])
    pltpu.sync_copy(degi_acc.at[pl.ds(s * RPT, RPT)],
                    out_hbm.at[c, 1, pl.ds(s * RPT, RPT)])


@functools.partial(
    pl.kernel,
    out_type=jax.ShapeDtypeStruct((NC, N, D), jnp.float32),
    mesh=_mesh,
    scratch_types=[
        pltpu.VMEM((HCH, ACH), jnp.int32),
        pltpu.VMEM((HCH, ACH), jnp.int32),
        pltpu.VMEM((2, ACH, D), jnp.float32),
        pltpu.VMEM_SHARED((NACC, D), jnp.float32),
        pltpu.SemaphoreType.DMA,
        pltpu.SemaphoreType.DMA,
        pltpu.SemaphoreType.DMA,
        pltpu.SemaphoreType.DMA,
        pltpu.SemaphoreType.DMA,
    ],
)
def _agg_sc(table_hbm, src_hbm, dst_hbm, zeros_hbm, out_hbm, src_v, dst_v,
            rows_v, acc, sg0, sg1, ss0, ss1, si):
    c = lax.axis_index("c")
    s = lax.axis_index("s")
    w = c * NS + s
    sg = (sg0, sg1)
    ss = (ss0, ss1)

    hs = pltpu.async_copy(src_hbm.at[w, pl.ds(0, HCH), :], src_v, si)
    hd = pltpu.async_copy(dst_hbm.at[w, pl.ds(0, HCH), :], dst_v, si)
    base = s * RPA

    @pl.when(s < NS - 1)
    def _():
        pltpu.async_copy(zeros_hbm.at[pl.ds(base, RPA), :],
                         acc.at[pl.ds(base, RPA), :], si)

    @pl.when(s == NS - 1)
    def _():
        pltpu.async_copy(zeros_hbm.at[pl.ds(base, RPL), :],
                         acc.at[pl.ds(base, RPL), :], si)
    hs.wait()
    hd.wait()

    @pl.when(s < NS - 1)
    def _():
        pltpu.make_async_copy(zeros_hbm.at[pl.ds(base, RPA), :],
                              acc.at[pl.ds(base, RPA), :], si).wait()

    @pl.when(s == NS - 1)
    def _():
        pltpu.make_async_copy(zeros_hbm.at[pl.ds(base, RPL), :],
                              acc.at[pl.ds(base, RPL), :], si).wait()
    plsc.subcore_barrier()

    def gather(j, b):
        pltpu.async_copy(table_hbm.at[src_v.at[j]], rows_v.at[b], sg[b])

    def gather_wait(j, b):
        pltpu.make_async_copy(table_hbm.at[src_v.at[j]], rows_v.at[b],
                              sg[b]).wait()

    def scatter(j, b):
        pltpu.async_copy(rows_v.at[b], acc.at[dst_v.at[j]], ss[b],
                         add=True)

    def scatter_wait(j, b):
        pltpu.make_async_copy(rows_v.at[b], acc.at[dst_v.at[j]],
                              ss[b]).wait()

    for p in range(2):
        if p == 1:
            pltpu.sync_copy(src_hbm.at[w, pl.ds(HCH, HCH), :], src_v)
            pltpu.sync_copy(dst_hbm.at[w, pl.ds(HCH, HCH), :], dst_v)
        gather(0, 0)
        gather_wait(0, 0)
        scatter(0, 0)
        gather(1, 1)

        def group(g, carry):
            for b in range(2):
                j = 2 * g + 1 + b
                gather_wait(j, 1 - b)
                scatter(j, 1 - b)
                scatter_wait(j - 1, b)
                gather(j + 1, b)
            return carry
        lax.fori_loop(0, (HCH - 2) // 2, group, 0)
        gather_wait(HCH - 1, 1)
        scatter(HCH - 1, 1)
        scatter_wait(HCH - 2, 0)
        scatter_wait(HCH - 1, 1)
    plsc.subcore_barrier()

    @pl.when(s < NS - 1)
    def _():
        pltpu.sync_copy(acc.at[pl.ds(s * RPA, RPA), :],
                        out_hbm.at[c, pl.ds(s * RPA, RPA), :])

    @pl.when(s == NS - 1)
    def _():
        pltpu.sync_copy(acc.at[pl.ds(s * RPA, CPL), :],
                        out_hbm.at[c, pl.ds(s * RPA, CPL), :])


_B = 1024


def _rs(col):
    return lax.rsqrt(jnp.maximum(col, 1.0))


def _deg_cols(dg_ref, i):
    dsl = pl.ds(i * _B, _B)
    d_out = (dg_ref[0, 0, dsl] + dg_ref[1, 0, dsl]).reshape(1, _B)
    d_in = (dg_ref[0, 1, dsl] + dg_ref[1, 1, dsl]).reshape(1, _B)
    return jnp.transpose(d_out), jnp.transpose(d_in)


def _t1_body(x_ref, w_ref, b_ref, dg_ref, o_ref):
    d_out, _ = _deg_cols(dg_ref, pl.program_id(0))
    rs_out = _rs(d_out)
    o_ref[...] = (jnp.dot(x_ref[...], w_ref[...],
                          preferred_element_type=jnp.float32)
                  + b_ref[...]) * rs_out


def _t2_body(a_ref, w_ref, b_ref, dg_ref, o_ref):
    d_out, d_in = _deg_cols(dg_ref, pl.program_id(0))
    rs_in = _rs(d_in)
    rs_out = _rs(d_out)
    h = jnp.maximum((a_ref[0] + a_ref[1]) * rs_in, 0.0)
    o_ref[...] = (jnp.dot(h, w_ref[...],
                          preferred_element_type=jnp.float32)
                  + b_ref[...]) * rs_out


def _t3_body(a_ref, dg_ref, f1_ref, g1_ref, f2_ref, g2_ref, f3_ref, g3_ref,
             fs_ref, gs_ref, o_ref):
    _, d_in = _deg_cols(dg_ref, pl.program_id(0))
    rs_in = _rs(d_in)
    h = (a_ref[0] + a_ref[1]) * rs_in
    ff = jnp.maximum(jnp.dot(h, f1_ref[...],
                             preferred_element_type=jnp.float32)
                     + g1_ref[...], 0.0)
    ff = jnp.maximum(jnp.dot(ff, f2_ref[...],
                             preferred_element_type=jnp.float32)
                     + g2_ref[...], 0.0)
    ff = jnp.maximum(jnp.dot(ff, f3_ref[...],
                             preferred_element_type=jnp.float32)
                     + g3_ref[...], 0.0)
    o_ref[...] = ff + jnp.dot(h, fs_ref[...],
                              preferred_element_type=jnp.float32) + gs_ref[...]


def _row_spec():
    return pl.BlockSpec((_B, D), lambda i: (i, 0))


def _full_spec(shape):
    return pl.BlockSpec(shape, lambda i: tuple(0 for _ in shape))


def _dg_spec():
    return pl.BlockSpec((NC, 2, NPAD), lambda i: (0, 0, 0))


def _agg_spec():
    return pl.BlockSpec((NC, _B, D), lambda i: (0, i, 0))


_GRID = ((N + _B - 1) // _B,)


def _t1(x, W, b, degT):
    return pl.pallas_call(
        _t1_body,
        grid=_GRID,
        in_specs=[_row_spec(), _full_spec((D, D)), _full_spec((1, D)),
                  _dg_spec()],
        out_specs=_row_spec(),
        out_shape=jax.ShapeDtypeStruct((N, D), jnp.float32),
    )(x, W, b, degT)


def _t2(a, W, b, degT):
    return pl.pallas_call(
        _t2_body,
        grid=_GRID,
        in_specs=[_agg_spec(), _full_spec((D, D)), _full_spec((1, D)),
                  _dg_spec()],
        out_specs=_row_spec(),
        out_shape=jax.ShapeDtypeStruct((N, D), jnp.float32),
    )(a, W, b, degT)


def _t3(a, degT, F1, g1, F2, g2, F3, g3, Fs, gs):
    fw = _full_spec((D, D))
    fb = _full_spec((1, D))
    return pl.pallas_call(
        _t3_body,
        grid=_GRID,
        in_specs=[_agg_spec(), _dg_spec(), fw, fb, fw, fb, fw, fb, fw, fb],
        out_specs=_row_spec(),
        out_shape=jax.ShapeDtypeStruct((N, D), jnp.float32),
    )(a, degT, F1, g1, F2, g2, F3, g3, Fs, gs)


def kernel(x, edge_index, W1, b1, W2, b2, F1, fb1, F2, fb2, F3, fb3, Fs, fbs):
    src_d = edge_index[0].reshape(NW, CPW, CH)
    dst_d = edge_index[1].reshape(NW, CPW, CH)
    npd = EPAD - E
    pad_src = (jnp.arange(npd, dtype=jnp.int32) % N)
    pad_dst = N + (jnp.arange(npd, dtype=jnp.int32) % NTRASH)
    src_a = jnp.concatenate([edge_index[0], pad_src]).reshape(NW, ACPW, ACH)
    dst_a = jnp.concatenate([edge_index[1], pad_dst]).reshape(NW, ACPW, ACH)

    zrows = jnp.zeros((NACC, D), jnp.float32)

    deg = _deg_sc(src_d, dst_d)

    s1 = _t1(x, W1, b1.reshape(1, D), deg)
    a1 = _agg_sc(s1, src_a, dst_a, zrows)
    s2 = _t2(a1, W2, b2.reshape(1, D), deg)
    a2 = _agg_sc(s2, src_a, dst_a, zrows)
    return _t3(a2, deg, F1, fb1.reshape(1, D), F2, fb2.reshape(1, D),
               F3, fb3.reshape(1, D), Fs, fbs.reshape(1, D))


# speedup vs baseline: 1.0362x; 1.0170x over previous
import functools

import jax
import jax.numpy as jnp
from jax import lax
from jax.experimental import pallas as pl
from jax.experimental.pallas import tpu as pltpu
from jax.experimental.pallas import tpu_sc as plsc

N = 10000
E = 320000
D = 128

NC = 2
NS = 16
NW = NC * NS

CH = 80
CPW = 125

ACH = 128
ACPW = 80
HCH = ACPW // 2
EPAD = NW * ACPW * ACH
NTRASH = 480
NACC = N + NTRASH
RPA = 656
RPL = NACC - 15 * RPA
CPL = N - 15 * RPA

NPAD = 10240
RPT = NPAD // NS

_mesh = plsc.VectorSubcoreMesh(
    core_axis_name="c", subcore_axis_name="s", num_cores=NC, num_subcores=NS
)


@functools.partial(
    pl.kernel,
    out_type=jax.ShapeDtypeStruct((NC, 2, NPAD), jnp.float32),
    mesh=_mesh,
    scratch_types=[
        pltpu.VMEM((CPW, CH), jnp.int32),
        pltpu.VMEM((CPW, CH), jnp.int32),
        pltpu.VMEM((RPT,), jnp.float32),
        pltpu.VMEM((CH,), jnp.float32),
        pltpu.VMEM_SHARED((NPAD,), jnp.float32),
        pltpu.VMEM_SHARED((NPAD,), jnp.float32),
        pltpu.SemaphoreType.DMA,
        pltpu.SemaphoreType.DMA,
    ],
)
def _deg_sc(src_hbm, dst_hbm, out_hbm, src_v, dst_v, zbuf, ones_v,
            dego_acc, degi_acc, sd, si):
    c = lax.axis_index("c")
    s = lax.axis_index("s")
    w = c * NS + s
    z16 = jnp.zeros((16,), jnp.float32)
    o16 = jnp.ones((16,), jnp.float32)

    hs = pltpu.async_copy(src_hbm.at[w], src_v, si)
    hd = pltpu.async_copy(dst_hbm.at[w], dst_v, si)

    def zrow(i, carry):
        zbuf[pl.ds(i * 16, 16)] = z16
        return carry
    lax.fori_loop(0, RPT // 16, zrow, 0)
    for q in range(CH // 16):
        ones_v[pl.ds(q * 16, 16)] = o16

    pltpu.sync_copy(zbuf, dego_acc.at[pl.ds(s * RPT, RPT)])
    pltpu.sync_copy(zbuf, degi_acc.at[pl.ds(s * RPT, RPT)])
    hs.wait()
    hd.wait()
    plsc.subcore_barrier()

    def group(g, carry):
        for b in range(25):
            j = 25 * g + b
            pltpu.async_copy(ones_v, dego_acc.at[src_v.at[j]], sd, add=True)
            pltpu.async_copy(ones_v, degi_acc.at[dst_v.at[j]], sd, add=True)
        for b in range(25):
            j = 25 * g + b
            pltpu.make_async_copy(ones_v, dego_acc.at[src_v.at[j]],
                                  sd).wait()
            pltpu.make_async_copy(ones_v, degi_acc.at[dst_v.at[j]],
                                  sd).wait()
        return carry
    lax.fori_loop(0, CPW // 25, group, 0)
    plsc.subcore_barrier()

    pltpu.sync_copy(dego_acc.at[pl.ds(s * RPT, RPT)],
                    out_hbm.at[c, 0, pl.ds(s * RPT, RPT)])
    pltpu.sync_copy(degi_acc.at[pl.ds(s * RPT, RPT)],
                    out_hbm.at[c, 1, pl.ds(s * RPT, RPT)])


@functools.partial(
    pl.kernel,
    out_type=jax.ShapeDtypeStruct((NC, N, D), jnp.float32),
    mesh=_mesh,
    scratch_types=[
        pltpu.VMEM((HCH, ACH), jnp.int32),
        pltpu.VMEM((HCH, ACH), jnp.int32),
        pltpu.VMEM((2, ACH, D), jnp.float32),
        pltpu.VMEM_SHARED((NACC, D), jnp.float32),
        pltpu.SemaphoreType.DMA,
        pltpu.SemaphoreType.DMA,
        pltpu.SemaphoreType.DMA,
        pltpu.SemaphoreType.DMA,
        pltpu.SemaphoreType.DMA,
    ],
)
def _agg_sc(table_hbm, src_hbm, dst_hbm, zeros_hbm, out_hbm, src_v, dst_v,
            rows_v, acc, sg0, sg1, ss0, ss1, si):
    c = lax.axis_index("c")
    s = lax.axis_index("s")
    w = c * NS + s
    sg = (sg0, sg1)
    ss = (ss0, ss1)

    hs = pltpu.async_copy(src_hbm.at[w, pl.ds(0, HCH), :], src_v, si)
    hd = pltpu.async_copy(dst_hbm.at[w, pl.ds(0, HCH), :], dst_v, si)
    base = s * RPA

    @pl.when(s < NS - 1)
    def _():
        pltpu.async_copy(zeros_hbm.at[pl.ds(base, RPA), :],
                         acc.at[pl.ds(base, RPA), :], si)

    @pl.when(s == NS - 1)
    def _():
        pltpu.async_copy(zeros_hbm.at[pl.ds(base, RPL), :],
                         acc.at[pl.ds(base, RPL), :], si)
    hs.wait()
    hd.wait()

    @pl.when(s < NS - 1)
    def _():
        pltpu.make_async_copy(zeros_hbm.at[pl.ds(base, RPA), :],
                              acc.at[pl.ds(base, RPA), :], si).wait()

    @pl.when(s == NS - 1)
    def _():
        pltpu.make_async_copy(zeros_hbm.at[pl.ds(base, RPL), :],
                              acc.at[pl.ds(base, RPL), :], si).wait()
    plsc.subcore_barrier()

    def gather(j, b):
        pltpu.async_copy(table_hbm.at[src_v.at[j]], rows_v.at[b], sg[b])

    def gather_wait(j, b):
        pltpu.make_async_copy(table_hbm.at[src_v.at[j]], rows_v.at[b],
                              sg[b]).wait()

    def scatter(j, b):
        pltpu.async_copy(rows_v.at[b], acc.at[dst_v.at[j]], ss[b],
                         add=True)

    def scatter_wait(j, b):
        pltpu.make_async_copy(rows_v.at[b], acc.at[dst_v.at[j]],
                              ss[b]).wait()

    for p in range(2):
        if p == 1:
            pltpu.sync_copy(src_hbm.at[w, pl.ds(HCH, HCH), :], src_v)
            pltpu.sync_copy(dst_hbm.at[w, pl.ds(HCH, HCH), :], dst_v)
        gather(0, 0)
        gather_wait(0, 0)
        scatter(0, 0)
        gather(1, 1)

        def group(g, carry):
            for b in range(2):
                j = 2 * g + 1 + b
                gather_wait(j, 1 - b)
                scatter(j, 1 - b)
                scatter_wait(j - 1, b)
                gather(j + 1, b)
            return carry
        lax.fori_loop(0, (HCH - 2) // 2, group, 0)
        gather_wait(HCH - 1, 1)
        scatter(HCH - 1, 1)
        scatter_wait(HCH - 2, 0)
        scatter_wait(HCH - 1, 1)
    plsc.subcore_barrier()

    @pl.when(s < NS - 1)
    def _():
        pltpu.sync_copy(acc.at[pl.ds(s * RPA, RPA), :],
                        out_hbm.at[c, pl.ds(s * RPA, RPA), :])

    @pl.when(s == NS - 1)
    def _():
        pltpu.sync_copy(acc.at[pl.ds(s * RPA, CPL), :],
                        out_hbm.at[c, pl.ds(s * RPA, CPL), :])


_B = 2048


def _rs(col):
    return lax.rsqrt(jnp.maximum(col, 1.0))


def _deg_cols(dg_ref, i):
    dsl = pl.ds(i * _B, _B)
    d_out = (dg_ref[0, 0, dsl] + dg_ref[1, 0, dsl]).reshape(1, _B)
    d_in = (dg_ref[0, 1, dsl] + dg_ref[1, 1, dsl]).reshape(1, _B)
    return jnp.transpose(d_out), jnp.transpose(d_in)


def _t1_body(x_ref, w_ref, b_ref, dg_ref, o_ref):
    d_out, _ = _deg_cols(dg_ref, pl.program_id(0))
    rs_out = _rs(d_out)
    o_ref[...] = (jnp.dot(x_ref[...], w_ref[...],
                          preferred_element_type=jnp.float32)
                  + b_ref[...]) * rs_out


def _t2_body(a_ref, w_ref, b_ref, dg_ref, o_ref):
    d_out, d_in = _deg_cols(dg_ref, pl.program_id(0))
    rs_in = _rs(d_in)
    rs_out = _rs(d_out)
    h = jnp.maximum((a_ref[0] + a_ref[1]) * rs_in, 0.0)
    o_ref[...] = (jnp.dot(h, w_ref[...],
                          preferred_element_type=jnp.float32)
                  + b_ref[...]) * rs_out


def _t3_body(a_ref, dg_ref, f1_ref, g1_ref, f2_ref, g2_ref, f3_ref, g3_ref,
             fs_ref, gs_ref, o_ref):
    _, d_in = _deg_cols(dg_ref, pl.program_id(0))
    rs_in = _rs(d_in)
    h = (a_ref[0] + a_ref[1]) * rs_in
    ff = jnp.maximum(jnp.dot(h, f1_ref[...],
                             preferred_element_type=jnp.float32)
                     + g1_ref[...], 0.0)
    ff = jnp.maximum(jnp.dot(ff, f2_ref[...],
                             preferred_element_type=jnp.float32)
                     + g2_ref[...], 0.0)
    ff = jnp.maximum(jnp.dot(ff, f3_ref[...],
                             preferred_element_type=jnp.float32)
                     + g3_ref[...], 0.0)
    o_ref[...] = ff + jnp.dot(h, fs_ref[...],
                              preferred_element_type=jnp.float32) + gs_ref[...]


def _row_spec():
    return pl.BlockSpec((_B, D), lambda i: (i, 0))


def _full_spec(shape):
    return pl.BlockSpec(shape, lambda i: tuple(0 for _ in shape))


def _dg_spec():
    return pl.BlockSpec((NC, 2, NPAD), lambda i: (0, 0, 0))


def _agg_spec():
    return pl.BlockSpec((NC, _B, D), lambda i: (0, i, 0))


_GRID = ((N + _B - 1) // _B,)


def _t1(x, W, b, degT):
    return pl.pallas_call(
        _t1_body,
        grid=_GRID,
        in_specs=[_row_spec(), _full_spec((D, D)), _full_spec((1, D)),
                  _dg_spec()],
        out_specs=_row_spec(),
        out_shape=jax.ShapeDtypeStruct((N, D), jnp.float32),
    )(x, W, b, degT)


def _t2(a, W, b, degT):
    return pl.pallas_call(
        _t2_body,
        grid=_GRID,
        in_specs=[_agg_spec(), _full_spec((D, D)), _full_spec((1, D)),
                  _dg_spec()],
        out_specs=_row_spec(),
        out_shape=jax.ShapeDtypeStruct((N, D), jnp.float32),
    )(a, W, b, degT)


def _t3(a, degT, F1, g1, F2, g2, F3, g3, Fs, gs):
    fw = _full_spec((D, D))
    fb = _full_spec((1, D))
    return pl.pallas_call(
        _t3_body,
        grid=_GRID,
        in_specs=[_agg_spec(), _dg_spec(), fw, fb, fw, fb, fw, fb, fw, fb],
        out_specs=_row_spec(),
        out_shape=jax.ShapeDtypeStruct((N, D), jnp.float32),
    )(a, degT, F1, g1, F2, g2, F3, g3, Fs, gs)


def kernel(x, edge_index, W1, b1, W2, b2, F1, fb1, F2, fb2, F3, fb3, Fs, fbs):
    src_d = edge_index[0].reshape(NW, CPW, CH)
    dst_d = edge_index[1].reshape(NW, CPW, CH)
    npd = EPAD - E
    pad_pair = jnp.stack([
        jnp.arange(npd, dtype=jnp.int32) % N,
        N + jnp.arange(npd, dtype=jnp.int32) % NTRASH,
    ])
    ea = jnp.concatenate([edge_index, pad_pair], axis=1)
    src_a = ea[0].reshape(NW, ACPW, ACH)
    dst_a = ea[1].reshape(NW, ACPW, ACH)

    zrows = jnp.zeros((NACC, D), jnp.float32)

    deg = _deg_sc(src_d, dst_d)

    s1 = _t1(x, W1, b1.reshape(1, D), deg)
    a1 = _agg_sc(s1, src_a, dst_a, zrows)
    s2 = _t2(a1, W2, b2.reshape(1, D), deg)
    a2 = _agg_sc(s2, src_a, dst_a, zrows)
    return _t3(a2, deg, F1, fb1.reshape(1, D), F2, fb2.reshape(1, D),
               F3, fb3.reshape(1, D), Fs, fbs.reshape(1, D))
